# baseline (speedup 1.0000x reference)
import jax
import jax.numpy as jnp
from jax import lax
from jax.experimental import pallas as pl
from jax.experimental.pallas import tpu as pltpu
from jax.experimental.pallas import tpu_sc as plsc

NUM_RELATION = 1000
DIM = 64
BATCH = 16384

NC = 2
NS = 16
LANES = 16
NW = NC * NS
BPW = BATCH // NW
NCHUNK = 4
CCOLS = BPW // NCHUNK
GPC = CCOLS // LANES


def _distmult_kernel(sub4_hbm, obj4_hbm, rela_hbm, diag_hbm, out_hbm,
                     sub_v, obj_v, rows_v, idx_v, out_v,
                     sem_s, sem_o, sem_r):
    wid = lax.axis_index("s") * NC + lax.axis_index("c")
    base = wid * BPW
    lane = lax.iota(jnp.int32, LANES)

    pltpu.sync_copy(rela_hbm.at[pl.ds(base, BPW)], idx_v)

    def start_chunk(c):
        b = c % 2
        bh = wid * NCHUNK + c
        cps = [
            pltpu.make_async_copy(
                diag_hbm.at[idx_v.at[pl.ds(c * CCOLS, CCOLS)]],
                rows_v.at[b], sem_r),
        ]
        for dh in range(8):
            cps.append(pltpu.make_async_copy(
                sub4_hbm.at[dh, bh], sub_v.at[b, pl.ds(dh * 8, 8)], sem_s))
            cps.append(pltpu.make_async_copy(
                obj4_hbm.at[dh, bh], obj_v.at[b, pl.ds(dh * 8, 8)], sem_o))
        for cp in cps:
            cp.start()
        return cps

    cps = start_chunk(0)

    for c in range(NCHUNK):
        for cp in cps:
            cp.wait()
        if c + 1 < NCHUNK:
            cps = start_chunk(c + 1)

        b = c % 2
        sref = sub_v.at[b]
        oref = obj_v.at[b]
        rref = rows_v.at[b]

        def g_body(g, carry):
            colv = g * LANES + lane
            accs = [jnp.zeros((LANES,), jnp.float32) for _ in range(4)]
            for d in range(DIM):
                drow = (lane + d) & (DIM - 1)
                s = plsc.load_gather(sref, [drow, colv])
                t = plsc.load_gather(rref, [colv, drow])
                o = plsc.load_gather(oref, [drow, colv])
                accs[d % 4] = accs[d % 4] + s * t * o
            out_v[pl.ds(c * CCOLS + g * LANES, LANES)] = (
                (accs[0] + accs[1]) + (accs[2] + accs[3]))
            return carry

        lax.fori_loop(0, GPC, g_body, 0, unroll=False)

    pltpu.sync_copy(out_v, out_hbm.at[pl.ds(base, BPW)])


@jax.jit
def kernel(sub_embed, obj_embed, rela, diag):
    mesh = plsc.VectorSubcoreMesh(core_axis_name="c", subcore_axis_name="s")
    run = pl.kernel(
        _distmult_kernel,
        out_type=jax.ShapeDtypeStruct((BATCH,), jnp.float32),
        mesh=mesh,
        scratch_types=[
            pltpu.VMEM((2, DIM, CCOLS), jnp.float32),
            pltpu.VMEM((2, DIM, CCOLS), jnp.float32),
            pltpu.VMEM((2, CCOLS, DIM), jnp.float32),
            pltpu.VMEM((BPW,), jnp.int32),
            pltpu.VMEM((BPW,), jnp.float32),
            pltpu.SemaphoreType.DMA,
            pltpu.SemaphoreType.DMA,
            pltpu.SemaphoreType.DMA,
        ],
        compiler_params=pltpu.CompilerParams(
            needs_layout_passes=False, use_tc_tiling_on_sc=False),
    )
    def as_tiles(x):
        return x.T.reshape(8, 8, 128, 128).transpose(0, 2, 1, 3)

    return run(as_tiles(sub_embed), as_tiles(obj_embed),
               rela.astype(jnp.int32), diag)


# speedup vs baseline: 1.2999x; 1.2999x over previous
import jax
import jax.numpy as jnp
from jax import lax
from jax.experimental import pallas as pl
from jax.experimental.pallas import tpu as pltpu
from jax.experimental.pallas import tpu_sc as plsc

NUM_RELATION = 1000
DIM = 64
BATCH = 16384

NC = 2
NS = 16
LANES = 16
NW = NC * NS
BPW = BATCH // NW
NCHUNK = 4
CCOLS = BPW // NCHUNK
GPC = CCOLS // LANES


def _distmult_kernel(sub4_hbm, obj4_hbm, rela_hbm, diag_hbm, out_hbm,
                     sub_v, obj_v, rows_v, idx_v, out_v,
                     sem_s, sem_o, sem_r):
    wid = lax.axis_index("s") * NC + lax.axis_index("c")
    base = wid * BPW
    lane = lax.iota(jnp.int32, LANES)

    pltpu.sync_copy(rela_hbm.at[pl.ds(base, BPW)], idx_v)

    def start_chunk(c):
        b = c % 2
        bh = wid * NCHUNK + c
        cps = [
            pltpu.make_async_copy(
                diag_hbm.at[idx_v.at[pl.ds(c * CCOLS, CCOLS)]],
                rows_v.at[b], sem_r),
        ]
        for dh in range(8):
            cps.append(pltpu.make_async_copy(
                sub4_hbm.at[dh, bh], sub_v.at[b, pl.ds(dh * 8, 8)], sem_s))
            cps.append(pltpu.make_async_copy(
                obj4_hbm.at[dh, bh], obj_v.at[b, pl.ds(dh * 8, 8)], sem_o))
        for cp in cps:
            cp.start()
        return cps

    cps = start_chunk(0)

    for c in range(NCHUNK):
        for cp in cps:
            cp.wait()
        if c + 1 < NCHUNK:
            cps = start_chunk(c + 1)

        b = c % 2
        sref = sub_v.at[b]
        oref = obj_v.at[b]
        rref = rows_v.at[b]

        def g_body(g, carry):
            colv = g * LANES + lane
            acc0 = jnp.zeros((LANES,), jnp.float32)
            acc1 = jnp.zeros((LANES,), jnp.float32)
            for d in range(DIM):
                drow = (lane + d) & (DIM - 1)
                s = plsc.load_gather(sref, [drow, colv])
                t = plsc.load_gather(rref, [colv, drow])
                o = plsc.load_gather(oref, [drow, colv])
                p = s * t * o
                if d % 2 == 0:
                    acc0 = acc0 + p
                else:
                    acc1 = acc1 + p
            out_v[pl.ds(c * CCOLS + g * LANES, LANES)] = acc0 + acc1
            return carry

        lax.fori_loop(0, GPC, g_body, 0, unroll=2)

    pltpu.sync_copy(out_v, out_hbm.at[pl.ds(base, BPW)])


@jax.jit
def kernel(sub_embed, obj_embed, rela, diag):
    mesh = plsc.VectorSubcoreMesh(core_axis_name="c", subcore_axis_name="s")
    run = pl.kernel(
        _distmult_kernel,
        out_type=jax.ShapeDtypeStruct((BATCH,), jnp.float32),
        mesh=mesh,
        scratch_types=[
            pltpu.VMEM((2, DIM, CCOLS), jnp.float32),
            pltpu.VMEM((2, DIM, CCOLS), jnp.float32),
            pltpu.VMEM((2, CCOLS, DIM), jnp.float32),
            pltpu.VMEM((BPW,), jnp.int32),
            pltpu.VMEM((BPW,), jnp.float32),
            pltpu.SemaphoreType.DMA,
            pltpu.SemaphoreType.DMA,
            pltpu.SemaphoreType.DMA,
        ],
        compiler_params=pltpu.CompilerParams(
            needs_layout_passes=False, use_tc_tiling_on_sc=False),
    )
    def as_tiles(x):
        return x.T.reshape(8, 8, 128, 128).transpose(0, 2, 1, 3)

    return run(as_tiles(sub_embed), as_tiles(obj_embed),
               rela.astype(jnp.int32), diag)
